# Initial kernel scaffold; baseline (speedup 1.0000x reference)
import jax
import jax.numpy as jnp
from jax.experimental import pallas as pl


def kernel(x, edge_index, W1u, W1d, W1b, g1u, g1d, W2u, W2d, W2b, g2u, g2d, W3u, W3d, W3b, g3u, g3d):
    raise NotImplementedError("write your pallas kernel here")


# speedup vs baseline: 4.9547x; 4.9547x over previous
import functools

import jax
import jax.numpy as jnp
from jax import lax
from jax.experimental import pallas as pl
from jax.experimental.pallas import tpu as pltpu
from jax.experimental.pallas import tpu_sc as plsc

N = 10000
E = 160000
NGE = 120000
NDE = 40000
DH = 256
HALF = 128
EPS = 1e-12

NC = 2
NS = 16
CHUNK = 128
CPT = -(-E // (NS * CHUNK))
E_PAD = NS * CPT * CHUNK
RPT = N // NS
NDEG = 10240
BN = 1000

_mesh = functools.partial(
    plsc.VectorSubcoreMesh,
    core_axis_name="c", subcore_axis_name="s", num_cores=NC, num_subcores=NS)


def _splat(vec16, lane):
    idx = jnp.full((16, 1), lane, jnp.int32)
    dnums = lax.GatherDimensionNumbers(
        offset_dims=(), collapsed_slice_dims=(0,), start_index_map=(0,))
    return lax.gather(vec16, idx, dnums, (1,),
                      mode=lax.GatherScatterMode.PROMISE_IN_BOUNDS)


def _deg_body(didx, dout, dst_v, ones_v, zb_v, accum, sem):
    c = lax.axis_index("c")
    s = lax.axis_index("s")
    o16 = jnp.ones((16,), jnp.float32)
    z16 = jnp.zeros((16,), jnp.float32)
    for k in range(CHUNK // 16):
        ones_v[pl.ds(16 * k, 16)] = o16

    def zf(k, _):
        zb_v[pl.ds(16 * k, 16)] = z16
        return 0
    lax.fori_loop(0, 40, zf, 0)
    pltpu.sync_copy(zb_v, accum.at[pl.ds(s * 640, 640)])
    plsc.subcore_barrier()
    pltpu.sync_copy(didx.at[c * NS + s], dst_v)

    def chunk(j, _):
        pltpu.sync_copy(ones_v, accum.at[dst_v.at[j]], add=True)
        return 0
    lax.fori_loop(0, CPT, chunk, 0)
    plsc.subcore_barrier()
    pltpu.sync_copy(accum.at[pl.ds(s * 624, 624)], zb_v.at[pl.ds(0, 624)])
    pltpu.sync_copy(zb_v.at[pl.ds(0, 624)], dout.at[pl.ds(c * N + s * 624, 624)])

    @pl.when(s == NS - 1)
    def _():
        pltpu.sync_copy(accum.at[pl.ds(15 * 624, 640)], zb_v)
        pltpu.sync_copy(zb_v, dout.at[pl.ds(c * N + 15 * 624, 640)])


def _sc_degrees(didx):
    f = pl.kernel(
        _deg_body,
        out_type=jax.ShapeDtypeStruct((2 * N,), jnp.float32),
        mesh=_mesh(),
        scratch_types=[
            pltpu.VMEM((CPT, CHUNK), jnp.int32),
            pltpu.VMEM((CHUNK,), jnp.float32),
            pltpu.VMEM((640,), jnp.float32),
            pltpu.VMEM_SHARED((NDEG,), jnp.float32),
            pltpu.SemaphoreType.DMA,
        ])
    return f(didx)


_ROW_CHUNKS = ((0, 128), (128, 128), (256, 128), (384, 128), (512, 112))
_TAIL = (9984, 16)


def _fill_zeros(rows_v):
    z16 = jnp.zeros((16,), jnp.float32)

    def zf(i, _):
        rows_v[i // 8, pl.ds((i % 8) * 16, 16)] = z16
        return 0
    lax.fori_loop(0, CHUNK * HALF // 16, zf, 0)


def _msg_pass(tab, src_hbm, dst_hbm, w_hbm, out, c, s,
              src_v, dst_v, w_v, rows_v, accum, sem):
    _fill_zeros(rows_v)
    for off, sz in _ROW_CHUNKS:
        pltpu.sync_copy(rows_v.at[pl.ds(0, sz)],
                        accum.at[pl.ds(s * 624 + off, sz)])

    @pl.when(s == NS - 1)
    def _():
        pltpu.sync_copy(rows_v.at[pl.ds(0, _TAIL[1])],
                        accum.at[pl.ds(_TAIL[0], _TAIL[1])])
    pltpu.sync_copy(src_hbm.at[c * NS + s], src_v)
    pltpu.sync_copy(dst_hbm.at[s], dst_v)
    pltpu.sync_copy(w_hbm.at[s], w_v)
    plsc.subcore_barrier()

    def chunk(j, _):
        pltpu.async_copy(tab.at[src_v.at[j]], rows_v, sem).wait()

        def grp(g, _):
            w16 = w_v[j, pl.ds(g * 16, 16)]
            for l in range(16):
                spl = _splat(w16, l)
                i = g * 16 + l
                for v in range(HALF // 16):
                    sl = pl.ds(v * 16, 16)
                    rows_v[i, sl] = rows_v[i, sl] * spl
            return 0
        lax.fori_loop(0, CHUNK // 16, grp, 0)
        pltpu.sync_copy(rows_v, accum.at[dst_v.at[j]], add=True)
        return 0
    lax.fori_loop(0, CPT, chunk, 0)
    plsc.subcore_barrier()
    for off, sz in _ROW_CHUNKS:
        stg = rows_v.at[pl.ds(0, sz)]
        pltpu.sync_copy(accum.at[pl.ds(s * 624 + off, sz)], stg)
        pltpu.sync_copy(stg, out.at[pl.ds(c * N + s * 624 + off, sz)])

    @pl.when(s == NS - 1)
    def _():
        stg = rows_v.at[pl.ds(0, _TAIL[1])]
        pltpu.sync_copy(accum.at[pl.ds(_TAIL[0], _TAIL[1])], stg)
        pltpu.sync_copy(stg, out.at[pl.ds(c * N + _TAIL[0], _TAIL[1])])


def _msg_body(usrc, udst, uw, dsrc, ddst, dw, utab, dtab, out_up, out_dn,
              src_v, dst_v, w_v, rows_v, accum, sem):
    c = lax.axis_index("c")
    s = lax.axis_index("s")
    _msg_pass(utab, usrc, udst, uw, out_up, c, s,
              src_v, dst_v, w_v, rows_v, accum, sem)
    plsc.subcore_barrier()
    _msg_pass(dtab, dsrc, ddst, dw, out_dn, c, s,
              src_v, dst_v, w_v, rows_v, accum, sem)


def _sc_messages(usrc, udst, uw, dsrc, ddst, dw, utab, dtab):
    f = pl.kernel(
        _msg_body,
        out_type=[jax.ShapeDtypeStruct((2 * N, HALF), jnp.float32),
                  jax.ShapeDtypeStruct((2 * N, HALF), jnp.float32)],
        mesh=_mesh(),
        scratch_types=[
            pltpu.VMEM((CPT, CHUNK), jnp.int32),
            pltpu.VMEM((CPT, CHUNK), jnp.int32),
            pltpu.VMEM((CPT, CHUNK), jnp.float32),
            pltpu.VMEM((CHUNK, HALF), jnp.float32),
            pltpu.VMEM_SHARED((N, HALF), jnp.float32),
            pltpu.SemaphoreType.DMA,
        ])
    return f(usrc, udst, uw, dsrc, ddst, dw, utab, dtab)


def _leaky(v):
    return jnp.where(v >= 0, v, 0.1 * v)


def _store_split(out, uo, do_, bo):
    uo[0] = out[:, 0:HALF]
    uo[1] = out[:, HALF:2 * HALF]
    do_[0] = out[:, 2 * HALF:3 * HALF]
    do_[1] = out[:, 3 * HALF:4 * HALF]
    bo[...] = out[:, 4 * HALF:]


def _mm1_body(x_ref, w_ref, uo, do_, bo):
    out = jnp.dot(x_ref[...], w_ref[...], preferred_element_type=jnp.float32)
    _store_split(out, uo, do_, bo)


def _normed_cat(u_ref, d_ref, b_ref, dc_ref, dr_ref):
    dic = dc_ref[...]
    dic = jnp.where(dic > 0, 1.0 / dic, 0.0)
    dirv = dr_ref[...]
    dirv = jnp.where(dirv > 0, 1.0 / dirv, 0.0)
    u0 = u_ref[0] * dic
    u1 = u_ref[1] * dic
    d0 = d_ref[0] * dirv
    d1 = d_ref[1] * dirv
    b = b_ref[...]
    ss = (jnp.sum(u0 * u0, 1, keepdims=True) + jnp.sum(u1 * u1, 1, keepdims=True)
          + jnp.sum(d0 * d0, 1, keepdims=True) + jnp.sum(d1 * d1, 1, keepdims=True)
          + jnp.sum(b * b, 1, keepdims=True))
    inv = 1.0 / jnp.maximum(jnp.sqrt(ss), EPS)
    cat = jnp.concatenate([u0, u1, d0, d1, b], axis=1)
    return _leaky(cat * inv)


def _layer_body(u_ref, d_ref, b_ref, dc_ref, dr_ref, w_ref, uo, do_, bo):
    h = _normed_cat(u_ref, d_ref, b_ref, dc_ref, dr_ref)
    out = jnp.dot(h, w_ref[...], preferred_element_type=jnp.float32)
    _store_split(out, uo, do_, bo)


def _final_body(u_ref, d_ref, b_ref, dc_ref, dr_ref, web_ref):
    web_ref[...] = _normed_cat(u_ref, d_ref, b_ref, dc_ref, dr_ref)


def _mean_body(a1, a2, a3, b1, b2, b3, mu, md):
    third = jnp.float32(1.0 / 3.0)
    mu[...] = (a1[...] + a2[...] + a3[...]) * third
    md[...] = (b1[...] + b2[...] + b3[...]) * third


_split_spec = pl.BlockSpec((2, BN, HALF), lambda i: (0, i, 0))
_bias_spec = pl.BlockSpec((BN, 2 * HALF), lambda i: (i, 0))
_deg_spec = pl.BlockSpec((BN, 1), lambda i: (i, 0))
_split_out = [jax.ShapeDtypeStruct((2, N, HALF), jnp.float32),
              jax.ShapeDtypeStruct((2, N, HALF), jnp.float32),
              jax.ShapeDtypeStruct((N, 2 * HALF), jnp.float32)]


def _tc_mm1(x, w):
    return pl.pallas_call(
        _mm1_body,
        grid=(N // BN,),
        in_specs=[pl.BlockSpec((BN, DH), lambda i: (i, 0)),
                  pl.BlockSpec((DH, 3 * DH), lambda i: (0, 0))],
        out_specs=[_split_spec, _split_spec, _bias_spec],
        out_shape=_split_out,
    )(x, w)


def _tc_layer(u, d, b, dc, dr, w):
    return pl.pallas_call(
        _layer_body,
        grid=(N // BN,),
        in_specs=[_split_spec, _split_spec, _bias_spec, _deg_spec, _deg_spec,
                  pl.BlockSpec((3 * DH, 3 * DH), lambda i: (0, 0))],
        out_specs=[_split_spec, _split_spec, _bias_spec],
        out_shape=_split_out,
    )(u, d, b, dc, dr, w)


def _tc_final(u, d, b, dc, dr):
    return pl.pallas_call(
        _final_body,
        grid=(N // BN,),
        in_specs=[_split_spec, _split_spec, _bias_spec, _deg_spec, _deg_spec],
        out_specs=pl.BlockSpec((BN, 3 * DH), lambda i: (i, 0)),
        out_shape=jax.ShapeDtypeStruct((N, 3 * DH), jnp.float32),
    )(u, d, b, dc, dr)


def _tc_mean(g1u, g2u, g3u, g1d, g2d, g3d):
    sh = (750, 160)
    args = [a.reshape(sh) for a in (g1u, g2u, g3u, g1d, g2d, g3d)]
    mu, md = pl.pallas_call(
        _mean_body,
        out_shape=[jax.ShapeDtypeStruct(sh, jnp.float32)] * 2,
    )(*args)
    return mu.reshape(NGE), md.reshape(NGE)


def kernel(x, edge_index, W1u, W1d, W1b, g1u, g1d, W2u, W2d, W2b, g2u, g2d,
           W3u, W3d, W3b, g3u, g3d):
    row = edge_index[0]
    col = edge_index[1]
    pad = E_PAD - E
    pad_idx = (jnp.arange(pad, dtype=jnp.int32) * 37) % N
    rowp = jnp.concatenate([row, pad_idx]).reshape(NS, CPT, CHUNK)
    colp = jnp.concatenate([col, pad_idx]).reshape(NS, CPT, CHUNK)

    def mk_src(a):
        return jnp.concatenate([a, a + N], axis=0)

    usrc = mk_src(rowp)
    udst = colp
    dsrc = mk_src(colp)
    ddst = rowp

    dummy = N + (jnp.arange(pad, dtype=jnp.int32) % 240)
    cold = jnp.concatenate([col, dummy]).reshape(NS, CPT, CHUNK)
    rowd = jnp.concatenate([row, dummy]).reshape(NS, CPT, CHUNK)
    didx = jnp.concatenate([cold, rowd], axis=0)

    zpad = jnp.zeros((pad,), jnp.float32)
    ones_nde = jnp.ones((NDE,), jnp.float32)

    def mk_w(g):
        return jnp.concatenate([g, ones_nde, zpad]).reshape(NS, CPT, CHUNK)

    deg = _sc_degrees(didx)
    dc = deg[:N].reshape(N, 1)
    dr = deg[N:].reshape(N, 1)

    w1 = jnp.concatenate([W1u, W1d, W1b], axis=1)
    w2 = jnp.concatenate([W2u, W2d, W2b], axis=1)
    w3 = jnp.concatenate([W3u, W3d, W3b], axis=1)

    u, d, b = _tc_mm1(x, w1)
    for wl, gu, gd in ((w2, g1u, g1d), (w3, g2u, g2d), (None, g3u, g3d)):
        xu, xd = _sc_messages(usrc, udst, mk_w(gu), dsrc, ddst, mk_w(gd),
                              u.reshape(2 * N, HALF), d.reshape(2 * N, HALF))
        xu = xu.reshape(2, N, HALF)
        xd = xd.reshape(2, N, HALF)
        if wl is None:
            web_x = _tc_final(xu, xd, b, dc, dr)
        else:
            u, d, b = _tc_layer(xu, xd, b, dc, dr, wl)

    mu, md = _tc_mean(g1u, g2u, g3u, g1d, g2d, g3d)
    mean_up = jnp.concatenate([mu, ones_nde])
    mean_down = jnp.concatenate([md, ones_nde])
    return web_x, mean_up, mean_down


# speedup vs baseline: 7.5996x; 1.5338x over previous
import functools

import jax
import jax.numpy as jnp
from jax import lax
from jax.experimental import pallas as pl
from jax.experimental.pallas import tpu as pltpu
from jax.experimental.pallas import tpu_sc as plsc

N = 10000
E = 160000
NGE = 120000
NDE = 40000
DH = 256
HALF = 128
EPS = 1e-12

NC = 2
NS = 16
CHUNK = 128
CPT = 80
E_PAD = NS * CPT * CHUNK
RPT = N // NS
NDEG = 10240
BN = 1000

_mesh = functools.partial(
    plsc.VectorSubcoreMesh,
    core_axis_name="c", subcore_axis_name="s", num_cores=NC, num_subcores=NS)


def _splat(vec16, lane):
    idx = jnp.full((16, 1), lane, jnp.int32)
    dnums = lax.GatherDimensionNumbers(
        offset_dims=(), collapsed_slice_dims=(0,), start_index_map=(0,))
    return lax.gather(vec16, idx, dnums, (1,),
                      mode=lax.GatherScatterMode.PROMISE_IN_BOUNDS)


def _deg_body(didx, dout, dst_v, ones_v, zb_v, accum, sem):
    c = lax.axis_index("c")
    s = lax.axis_index("s")
    o16 = jnp.ones((16,), jnp.float32)
    z16 = jnp.zeros((16,), jnp.float32)
    for k in range(CHUNK // 16):
        ones_v[pl.ds(16 * k, 16)] = o16

    def zf(k, _):
        zb_v[pl.ds(16 * k, 16)] = z16
        return 0
    lax.fori_loop(0, 40, zf, 0)
    pltpu.sync_copy(zb_v, accum.at[pl.ds(s * 640, 640)])
    plsc.subcore_barrier()
    pltpu.sync_copy(didx.at[c * NS + s], dst_v)

    def chunk(j, _):
        pltpu.sync_copy(ones_v, accum.at[dst_v.at[j]], add=True)
        return 0
    lax.fori_loop(0, CPT, chunk, 0)
    plsc.subcore_barrier()
    pltpu.sync_copy(accum.at[pl.ds(s * 624, 624)], zb_v.at[pl.ds(0, 624)])
    pltpu.sync_copy(zb_v.at[pl.ds(0, 624)], dout.at[pl.ds(c * N + s * 624, 624)])

    @pl.when(s == NS - 1)
    def _():
        pltpu.sync_copy(accum.at[pl.ds(15 * 624, 640)], zb_v)
        pltpu.sync_copy(zb_v, dout.at[pl.ds(c * N + 15 * 624, 640)])


def _sc_degrees(didx):
    f = pl.kernel(
        _deg_body,
        out_type=jax.ShapeDtypeStruct((2 * N,), jnp.float32),
        mesh=_mesh(),
        scratch_types=[
            pltpu.VMEM((CPT, CHUNK), jnp.int32),
            pltpu.VMEM((CHUNK,), jnp.float32),
            pltpu.VMEM((640,), jnp.float32),
            pltpu.VMEM_SHARED((NDEG,), jnp.float32),
            pltpu.SemaphoreType.DMA,
        ])
    return f(didx)


_ROW_CHUNKS = ((0, 128), (128, 128), (256, 128), (384, 128), (512, 112))
_TAIL = (9984, 16)


def _fill_zeros(rows_v):
    z16 = jnp.zeros((16,), jnp.float32)

    def zf(i, _):
        rows_v[i // 8, pl.ds((i % 8) * 16, 16)] = z16
        return 0
    lax.fori_loop(0, CHUNK * HALF // 16, zf, 0)


def _scale_rows(buf, w_v, j):
    def grp(g, _):
        w16 = w_v[j, pl.ds(g * 16, 16)]
        for l in range(16):
            spl = _splat(w16, l)
            i = g * 16 + l
            for v in range(HALF // 16):
                sl = pl.ds(v * 16, 16)
                buf[i, sl] = buf[i, sl] * spl
        return 0
    lax.fori_loop(0, CHUNK // 16, grp, 0)


BLK = 20
NBLK = CPT // BLK


def _msg_pass(tab, src_hbm, dst_hbm, w_hbm, out, c, s,
              src_v, dstb_v, wb_v, rows, accum, gs, ss):
    _fill_zeros(rows[0])
    for off, sz in _ROW_CHUNKS:
        pltpu.sync_copy(rows[0].at[pl.ds(0, sz)],
                        accum.at[pl.ds(s * 624 + off, sz)])

    @pl.when(s == NS - 1)
    def _():
        pltpu.sync_copy(rows[0].at[pl.ds(0, _TAIL[1])],
                        accum.at[pl.ds(_TAIL[0], _TAIL[1])])
    pltpu.sync_copy(src_hbm.at[c * NS + s], src_v)
    pltpu.sync_copy(dst_hbm.at[s * NBLK], dstb_v)
    pltpu.sync_copy(w_hbm.at[s * NBLK], wb_v)
    plsc.subcore_barrier()

    pltpu.async_copy(tab.at[src_v.at[0]], rows[0], gs[0])

    def pipe(k, _):
        for b in (0, 1):
            j = 2 * k + b
            jj = lax.rem(j, BLK)
            buf, other = rows[b], rows[1 - b]

            def drain():
                pltpu.make_async_copy(
                    other, accum.at[dstb_v.at[lax.rem(j - 1, BLK)]],
                    ss[1 - b]).wait()
            if b == 0:
                pl.when(k > 0)(drain)

                def restage():
                    blk = k // (BLK // 2)
                    pltpu.sync_copy(dst_hbm.at[s * NBLK + blk], dstb_v)
                    pltpu.sync_copy(w_hbm.at[s * NBLK + blk], wb_v)
                pl.when((k > 0) & (lax.rem(k, BLK // 2) == 0))(restage)
                pltpu.async_copy(tab.at[src_v.at[j + 1]], other, gs[1 - b])
            else:
                drain()

                def nxt():
                    pltpu.async_copy(tab.at[src_v.at[j + 1]], other, gs[1 - b])
                pl.when(k < CPT // 2 - 1)(nxt)

            pltpu.make_async_copy(tab.at[src_v.at[j]], buf, gs[b]).wait()
            _scale_rows(buf, wb_v, jj)
            pltpu.async_copy(buf, accum.at[dstb_v.at[jj]], ss[b], add=True)
        return 0
    lax.fori_loop(0, CPT // 2, pipe, 0)
    pltpu.make_async_copy(
        rows[1], accum.at[dstb_v.at[lax.rem(CPT - 1, BLK)]], ss[1]).wait()
    plsc.subcore_barrier()
    for off, sz in _ROW_CHUNKS:
        stg = rows[0].at[pl.ds(0, sz)]
        pltpu.sync_copy(accum.at[pl.ds(s * 624 + off, sz)], stg)
        pltpu.sync_copy(stg, out.at[pl.ds(c * N + s * 624 + off, sz)])

    @pl.when(s == NS - 1)
    def _():
        stg = rows[1].at[pl.ds(0, _TAIL[1])]
        pltpu.sync_copy(accum.at[pl.ds(_TAIL[0], _TAIL[1])], stg)
        pltpu.sync_copy(stg, out.at[pl.ds(c * N + _TAIL[0], _TAIL[1])])


def _msg_body(usrc, udst, uw, dsrc, ddst, dw, utab, dtab, out_up, out_dn,
              src_v, dstb_v, wb_v, r0, r1, accum, g0, g1, s0, s1):
    c = lax.axis_index("c")
    s = lax.axis_index("s")
    rows = (r0, r1)
    gs = (g0, g1)
    ss = (s0, s1)
    _msg_pass(utab, usrc, udst, uw, out_up, c, s,
              src_v, dstb_v, wb_v, rows, accum, gs, ss)
    plsc.subcore_barrier()
    _msg_pass(dtab, dsrc, ddst, dw, out_dn, c, s,
              src_v, dstb_v, wb_v, rows, accum, gs, ss)


def _sc_messages(usrc, udst, uw, dsrc, ddst, dw, utab, dtab):
    f = pl.kernel(
        _msg_body,
        out_type=[jax.ShapeDtypeStruct((2 * N, HALF), jnp.float32),
                  jax.ShapeDtypeStruct((2 * N, HALF), jnp.float32)],
        mesh=_mesh(),
        scratch_types=[
            pltpu.VMEM((CPT, CHUNK), jnp.int32),
            pltpu.VMEM((BLK, CHUNK), jnp.int32),
            pltpu.VMEM((BLK, CHUNK), jnp.float32),
            pltpu.VMEM((CHUNK, HALF), jnp.float32),
            pltpu.VMEM((CHUNK, HALF), jnp.float32),
            pltpu.VMEM_SHARED((N, HALF), jnp.float32),
            pltpu.SemaphoreType.DMA,
            pltpu.SemaphoreType.DMA,
            pltpu.SemaphoreType.DMA,
            pltpu.SemaphoreType.DMA,
        ])
    return f(usrc, udst, uw, dsrc, ddst, dw, utab, dtab)


def _leaky(v):
    return jnp.where(v >= 0, v, 0.1 * v)


def _store_split(out, uo, do_, bo):
    uo[0] = out[:, 0:HALF]
    uo[1] = out[:, HALF:2 * HALF]
    do_[0] = out[:, 2 * HALF:3 * HALF]
    do_[1] = out[:, 3 * HALF:4 * HALF]
    bo[...] = out[:, 4 * HALF:]


def _mm1_body(x_ref, w_ref, uo, do_, bo):
    out = jnp.dot(x_ref[...], w_ref[...], preferred_element_type=jnp.float32)
    _store_split(out, uo, do_, bo)


def _normed_cat(u_ref, d_ref, b_ref, dc_ref, dr_ref):
    dic = dc_ref[...]
    dic = jnp.where(dic > 0, 1.0 / dic, 0.0)
    dirv = dr_ref[...]
    dirv = jnp.where(dirv > 0, 1.0 / dirv, 0.0)
    u0 = u_ref[0] * dic
    u1 = u_ref[1] * dic
    d0 = d_ref[0] * dirv
    d1 = d_ref[1] * dirv
    b = b_ref[...]
    ss = (jnp.sum(u0 * u0, 1, keepdims=True) + jnp.sum(u1 * u1, 1, keepdims=True)
          + jnp.sum(d0 * d0, 1, keepdims=True) + jnp.sum(d1 * d1, 1, keepdims=True)
          + jnp.sum(b * b, 1, keepdims=True))
    inv = 1.0 / jnp.maximum(jnp.sqrt(ss), EPS)
    cat = jnp.concatenate([u0, u1, d0, d1, b], axis=1)
    return _leaky(cat * inv)


def _layer_body(u_ref, d_ref, b_ref, dc_ref, dr_ref, w_ref, uo, do_, bo):
    h = _normed_cat(u_ref, d_ref, b_ref, dc_ref, dr_ref)
    out = jnp.dot(h, w_ref[...], preferred_element_type=jnp.float32)
    _store_split(out, uo, do_, bo)


def _final_body(u_ref, d_ref, b_ref, dc_ref, dr_ref, web_ref):
    web_ref[...] = _normed_cat(u_ref, d_ref, b_ref, dc_ref, dr_ref)


def _mean_body(a1, a2, a3, b1, b2, b3, mu, md):
    third = jnp.float32(1.0 / 3.0)
    mu[...] = (a1[...] + a2[...] + a3[...]) * third
    md[...] = (b1[...] + b2[...] + b3[...]) * third


_split_spec = pl.BlockSpec((2, BN, HALF), lambda i: (0, i, 0))
_bias_spec = pl.BlockSpec((BN, 2 * HALF), lambda i: (i, 0))
_deg_spec = pl.BlockSpec((BN, 1), lambda i: (i, 0))
_split_out = [jax.ShapeDtypeStruct((2, N, HALF), jnp.float32),
              jax.ShapeDtypeStruct((2, N, HALF), jnp.float32),
              jax.ShapeDtypeStruct((N, 2 * HALF), jnp.float32)]


def _tc_mm1(x, w):
    return pl.pallas_call(
        _mm1_body,
        grid=(N // BN,),
        in_specs=[pl.BlockSpec((BN, DH), lambda i: (i, 0)),
                  pl.BlockSpec((DH, 3 * DH), lambda i: (0, 0))],
        out_specs=[_split_spec, _split_spec, _bias_spec],
        out_shape=_split_out,
    )(x, w)


def _tc_layer(u, d, b, dc, dr, w):
    return pl.pallas_call(
        _layer_body,
        grid=(N // BN,),
        in_specs=[_split_spec, _split_spec, _bias_spec, _deg_spec, _deg_spec,
                  pl.BlockSpec((3 * DH, 3 * DH), lambda i: (0, 0))],
        out_specs=[_split_spec, _split_spec, _bias_spec],
        out_shape=_split_out,
    )(u, d, b, dc, dr, w)


def _tc_final(u, d, b, dc, dr):
    return pl.pallas_call(
        _final_body,
        grid=(N // BN,),
        in_specs=[_split_spec, _split_spec, _bias_spec, _deg_spec, _deg_spec],
        out_specs=pl.BlockSpec((BN, 3 * DH), lambda i: (i, 0)),
        out_shape=jax.ShapeDtypeStruct((N, 3 * DH), jnp.float32),
    )(u, d, b, dc, dr)


def _tc_mean(g1u, g2u, g3u, g1d, g2d, g3d):
    sh = (750, 160)
    args = [a.reshape(sh) for a in (g1u, g2u, g3u, g1d, g2d, g3d)]
    mu, md = pl.pallas_call(
        _mean_body,
        out_shape=[jax.ShapeDtypeStruct(sh, jnp.float32)] * 2,
    )(*args)
    return mu.reshape(NGE), md.reshape(NGE)


def kernel(x, edge_index, W1u, W1d, W1b, g1u, g1d, W2u, W2d, W2b, g2u, g2d,
           W3u, W3d, W3b, g3u, g3d):
    row = edge_index[0]
    col = edge_index[1]
    pad = E_PAD - E
    pad_idx = (jnp.arange(pad, dtype=jnp.int32) * 37) % N
    rowp = jnp.concatenate([row, pad_idx])
    colp = jnp.concatenate([col, pad_idx])

    def mk_src(a):
        a = a.reshape(NS, CPT, CHUNK)
        return jnp.concatenate([a, a + N], axis=0)

    def mk_dst(a):
        return a.reshape(NS * NBLK, BLK, CHUNK)

    usrc = mk_src(rowp)
    udst = mk_dst(colp)
    dsrc = mk_src(colp)
    ddst = mk_dst(rowp)

    dummy = N + (jnp.arange(pad, dtype=jnp.int32) % 240)
    cold = jnp.concatenate([col, dummy]).reshape(NS, CPT, CHUNK)
    rowd = jnp.concatenate([row, dummy]).reshape(NS, CPT, CHUNK)
    didx = jnp.concatenate([cold, rowd], axis=0)

    zpad = jnp.zeros((pad,), jnp.float32)
    ones_nde = jnp.ones((NDE,), jnp.float32)

    def mk_w(g):
        return jnp.concatenate([g, ones_nde, zpad]).reshape(NS * NBLK, BLK, CHUNK)

    deg = _sc_degrees(didx)
    dc = deg[:N].reshape(N, 1)
    dr = deg[N:].reshape(N, 1)

    w1 = jnp.concatenate([W1u, W1d, W1b], axis=1)
    w2 = jnp.concatenate([W2u, W2d, W2b], axis=1)
    w3 = jnp.concatenate([W3u, W3d, W3b], axis=1)

    u, d, b = _tc_mm1(x, w1)
    for wl, gu, gd in ((w2, g1u, g1d), (w3, g2u, g2d), (None, g3u, g3d)):
        xu, xd = _sc_messages(usrc, udst, mk_w(gu), dsrc, ddst, mk_w(gd),
                              u.reshape(2 * N, HALF), d.reshape(2 * N, HALF))
        xu = xu.reshape(2, N, HALF)
        xd = xd.reshape(2, N, HALF)
        if wl is None:
            web_x = _tc_final(xu, xd, b, dc, dr)
        else:
            u, d, b = _tc_layer(xu, xd, b, dc, dr, wl)

    mu, md = _tc_mean(g1u, g2u, g3u, g1d, g2d, g3d)
    mean_up = jnp.concatenate([mu, ones_nde])
    mean_down = jnp.concatenate([md, ones_nde])
    return web_x, mean_up, mean_down
